# baseline (speedup 1.0000x reference)
import jax
import jax.numpy as jnp
from jax.experimental import pallas as pl

N_NODES = 100000
N_EDGES = 1600000
IN_DIM = 131


def _mlp_body(x_ref, w1_ref, b1_ref, w2_ref, b2_ref, w3_ref, b3_ref, o_ref):
    h = jnp.tanh(jnp.dot(x_ref[...], w1_ref[...],
                         preferred_element_type=jnp.float32) + b1_ref[...])
    h = jnp.tanh(jnp.dot(h, w2_ref[...],
                         preferred_element_type=jnp.float32) + b2_ref[...])
    o_ref[...] = jnp.dot(h, w3_ref[...],
                         preferred_element_type=jnp.float32) + b3_ref[...]


def _mlp(x, W1, b1, W2, b2, W3, b3):
    BLK = 2000
    grid = (N_NODES // BLK,)
    full = lambda s: pl.BlockSpec(s, lambda i: (0,) * len(s))
    return pl.pallas_call(
        _mlp_body,
        grid=grid,
        in_specs=[
            pl.BlockSpec((BLK, IN_DIM), lambda i: (i, 0)),
            full((IN_DIM, 64)), full((64,)),
            full((64, 32)), full((32,)),
            full((32, 16)), full((16,)),
        ],
        out_specs=pl.BlockSpec((BLK, 16), lambda i: (i, 0)),
        out_shape=jax.ShapeDtypeStruct((N_NODES, 16), jnp.float32),
    )(x, W1, b1, W2, b2, W3, b3)


def kernel(x, edge_index, W1, b1, W2, b2, W3, b3, convW, convb, Wc, bc):
    loop = jnp.arange(N_NODES, dtype=edge_index.dtype)
    src = jnp.concatenate([edge_index[0], loop])
    dst = jnp.concatenate([edge_index[1], loop])
    deg = jax.ops.segment_sum(jnp.ones_like(src, dtype=jnp.float32), dst,
                              num_segments=N_NODES)
    dinv = jnp.where(deg > 0, jax.lax.rsqrt(jnp.maximum(deg, 1e-12)), 0.0)
    norm = dinv[src] * dinv[dst]

    def gcn_conv(h, W, b):
        h2 = h @ W
        msg = h2[src] * norm[:, None]
        agg = jax.ops.segment_sum(msg, dst, num_segments=N_NODES)
        return agg + b

    h = _mlp(x, W1, b1, W2, b2, W3, b3)
    for _ in range(5):
        h = jnp.tanh(gcn_conv(h, convW[0], convb[0]))
    for i in range(5):
        h = jnp.tanh(gcn_conv(h, convW[i + 1], convb[i + 1]))
    out = h @ Wc + bc
    return (out, h)


# speedup vs baseline: 10.6062x; 10.6062x over previous
import jax
import jax.numpy as jnp
from jax import lax
from jax.experimental import pallas as pl
from jax.experimental.pallas import tpu as pltpu
from jax.experimental.pallas import tpu_sc as plsc

N_NODES = 100000
N_PAD = 100096
N_EDGES = 1600000
IN_DIM = 131
F = 16

NC = 2
NS = 16
NW = NC * NS
WIN = 128
NWIN = N_EDGES // WIN
WPW = NWIN // NW
WREM = NWIN % NW
RPT = N_PAD // NS
ZB = RPT // 8

_mesh = plsc.VectorSubcoreMesh(core_axis_name="c", subcore_axis_name="s")
_sc_params = pltpu.CompilerParams(use_tc_tiling_on_sc=False)


def _zero_acc(acc_sh, zbuf, s):
    def zb(i, carry):
        zbuf[i] = jnp.zeros((F,), jnp.float32)
        return carry

    lax.fori_loop(0, ZB, zb, 0)
    for j in range(RPT // ZB):
        pltpu.sync_copy(zbuf, acc_sh.at[pl.ds(s * RPT + j * ZB, ZB)])


def _edge_range(wid):
    count = WPW + (wid < WREM).astype(jnp.int32)
    start = wid * WPW + jnp.minimum(wid, WREM)
    return start, count


def _writeback(acc_sh, p0_hbm, p1_hbm, c, s):
    @pl.when(c == 0)
    def _():
        pltpu.sync_copy(acc_sh.at[pl.ds(s * RPT, RPT)],
                        p0_hbm.at[pl.ds(s * RPT, RPT)])

    @pl.when(c == 1)
    def _():
        pltpu.sync_copy(acc_sh.at[pl.ds(s * RPT, RPT)],
                        p1_hbm.at[pl.ds(s * RPT, RPT)])


def _sc_degree_body(dst_hbm, p0_hbm, p1_hbm, acc_sh, zbuf, didx, ones_b):
    c = lax.axis_index("c")
    s = lax.axis_index("s")
    wid = s * NC + c

    _zero_acc(acc_sh, zbuf, s)

    def ob(i, carry):
        ones_b[i] = jnp.ones((F,), jnp.float32)
        return carry

    lax.fori_loop(0, WIN, ob, 0)
    plsc.subcore_barrier()

    start, count = _edge_range(wid)

    def body(g, carry):
        off = (start + g) * WIN
        pltpu.sync_copy(dst_hbm.at[pl.ds(off, WIN)], didx)
        pltpu.sync_copy(ones_b, acc_sh.at[didx], add=True)
        return carry

    lax.fori_loop(0, count, body, 0)
    plsc.subcore_barrier()
    _writeback(acc_sh, p0_hbm, p1_hbm, c, s)


def _sc_layer_body(g_hbm, src_hbm, dst_hbm, p0_hbm, p1_hbm,
                   acc_sh, zbuf, sidx, didx, rows):
    c = lax.axis_index("c")
    s = lax.axis_index("s")
    wid = s * NC + c

    _zero_acc(acc_sh, zbuf, s)
    plsc.subcore_barrier()

    start, count = _edge_range(wid)

    def body(g, carry):
        off = (start + g) * WIN
        pltpu.sync_copy(src_hbm.at[pl.ds(off, WIN)], sidx)
        pltpu.sync_copy(dst_hbm.at[pl.ds(off, WIN)], didx)
        pltpu.sync_copy(g_hbm.at[sidx], rows)
        pltpu.sync_copy(rows, acc_sh.at[didx], add=True)
        return carry

    lax.fori_loop(0, count, body, 0)
    plsc.subcore_barrier()
    _writeback(acc_sh, p0_hbm, p1_hbm, c, s)


def _sc_degree(dst):
    return pl.kernel(
        _sc_degree_body,
        out_type=(jax.ShapeDtypeStruct((N_PAD, F), jnp.float32),
                  jax.ShapeDtypeStruct((N_PAD, F), jnp.float32)),
        mesh=_mesh,
        compiler_params=_sc_params,
        scratch_types=[
            pltpu.VMEM_SHARED((N_PAD, F), jnp.float32),
            pltpu.VMEM((ZB, F), jnp.float32),
            pltpu.VMEM((WIN,), jnp.int32),
            pltpu.VMEM((WIN, F), jnp.float32),
        ],
    )(dst)


def _sc_layer(g, src, dst):
    return pl.kernel(
        _sc_layer_body,
        out_type=(jax.ShapeDtypeStruct((N_PAD, F), jnp.float32),
                  jax.ShapeDtypeStruct((N_PAD, F), jnp.float32)),
        mesh=_mesh,
        compiler_params=_sc_params,
        scratch_types=[
            pltpu.VMEM_SHARED((N_PAD, F), jnp.float32),
            pltpu.VMEM((ZB, F), jnp.float32),
            pltpu.VMEM((WIN,), jnp.int32),
            pltpu.VMEM((WIN,), jnp.int32),
            pltpu.VMEM((WIN, F), jnp.float32),
        ],
    )(g, src, dst)


_BLK = N_PAD // 16


def _full(s):
    return pl.BlockSpec(s, lambda i: (0,) * len(s))


def _rows(w=F):
    return pl.BlockSpec((_BLK, w), lambda i: (i, 0))


def _mlp_body(x_ref, w1_ref, b1_ref, w2_ref, b2_ref, w3_ref, b3_ref, o_ref):
    h = jnp.tanh(jnp.dot(x_ref[...], w1_ref[...],
                         preferred_element_type=jnp.float32) + b1_ref[...])
    h = jnp.tanh(jnp.dot(h, w2_ref[...],
                         preferred_element_type=jnp.float32) + b2_ref[...])
    o_ref[...] = jnp.dot(h, w3_ref[...],
                         preferred_element_type=jnp.float32) + b3_ref[...]


def _mlp(x, W1, b1, W2, b2, W3, b3):
    BLK = 2000
    return pl.pallas_call(
        _mlp_body,
        grid=(N_NODES // BLK,),
        in_specs=[
            pl.BlockSpec((BLK, IN_DIM), lambda i: (i, 0)),
            _full((IN_DIM, 64)), _full((64,)),
            _full((64, 32)), _full((32,)),
            _full((32, F)), _full((F,)),
        ],
        out_specs=pl.BlockSpec((BLK, F), lambda i: (i, 0)),
        out_shape=jax.ShapeDtypeStruct((N_NODES, F), jnp.float32),
    )(x, W1, b1, W2, b2, W3, b3)


def _tc_pre_body(d0, d1, h0, w, dinv_o, g_o):
    dinv = lax.rsqrt(d0[...] + d1[...] + 1.0)
    dinv_o[...] = dinv
    g_o[...] = dinv * jnp.dot(h0[...], w[...],
                              preferred_element_type=jnp.float32)


def _tc_pre(d0, d1, h0p, W0):
    return pl.pallas_call(
        _tc_pre_body,
        grid=(N_PAD // _BLK,),
        in_specs=[_rows(), _rows(), _rows(), _full((F, F))],
        out_specs=[_rows(), _rows()],
        out_shape=[jax.ShapeDtypeStruct((N_PAD, F), jnp.float32),
                   jax.ShapeDtypeStruct((N_PAD, F), jnp.float32)],
    )(d0, d1, h0p, W0)


def _tc_mid_body(p0, p1, g, dinv, b, w, g_o):
    agg = dinv[...] * (p0[...] + p1[...] + g[...])
    h = jnp.tanh(agg + b[...])
    g_o[...] = dinv[...] * jnp.dot(h, w[...],
                                   preferred_element_type=jnp.float32)


def _tc_mid(p0, p1, g, dinv, b, Wn):
    return pl.pallas_call(
        _tc_mid_body,
        grid=(N_PAD // _BLK,),
        in_specs=[_rows(), _rows(), _rows(), _rows(),
                  _full((F,)), _full((F, F))],
        out_specs=_rows(),
        out_shape=jax.ShapeDtypeStruct((N_PAD, F), jnp.float32),
    )(p0, p1, g, dinv, b, Wn)


def _tc_last_body(p0, p1, g, dinv, b, wc, bc, h_o, out_o):
    agg = dinv[...] * (p0[...] + p1[...] + g[...])
    h = jnp.tanh(agg + b[...])
    h_o[...] = h
    out_o[...] = jnp.dot(h, wc[...],
                         preferred_element_type=jnp.float32) + bc[...]


def _tc_last(p0, p1, g, dinv, b, Wc, bc):
    return pl.pallas_call(
        _tc_last_body,
        grid=(N_PAD // _BLK,),
        in_specs=[_rows(), _rows(), _rows(), _rows(),
                  _full((F,)), _full((F, 2)), _full((2,))],
        out_specs=[_rows(), _rows(2)],
        out_shape=[jax.ShapeDtypeStruct((N_PAD, F), jnp.float32),
                   jax.ShapeDtypeStruct((N_PAD, 2), jnp.float32)],
    )(p0, p1, g, dinv, b, Wc, bc)


def kernel(x, edge_index, W1, b1, W2, b2, W3, b3, convW, convb, Wc, bc):
    src = edge_index[0]
    dst = edge_index[1]

    d0, d1 = _sc_degree(dst)
    h0 = _mlp(x, W1, b1, W2, b2, W3, b3)
    h0p = jnp.pad(h0, ((0, N_PAD - N_NODES), (0, 0)))
    dinv, g = _tc_pre(d0, d1, h0p, convW[0])

    widx = lambda k: 0 if k < 5 else k - 4
    for k in range(10):
        p0, p1 = _sc_layer(g, src, dst)
        if k < 9:
            g = _tc_mid(p0, p1, g, dinv, convb[widx(k)], convW[widx(k + 1)])
        else:
            h, out = _tc_last(p0, p1, g, dinv, convb[widx(k)], Wc, bc)

    return (out[:N_NODES], h[:N_NODES])


# speedup vs baseline: 30.0194x; 2.8304x over previous
import jax
import jax.numpy as jnp
from jax import lax
from jax.experimental import pallas as pl
from jax.experimental.pallas import tpu as pltpu
from jax.experimental.pallas import tpu_sc as plsc

N_NODES = 100000
N_PAD = 100096
N_EDGES = 1600000
IN_DIM = 131
F = 16

NC = 2
NS = 16
NW = NC * NS
WIN = 128
NWIN = N_EDGES // WIN
WPW = NWIN // NW
WREM = NWIN % NW
RPT = N_PAD // NS
ZB = RPT // 16

NB = 8
NCH = 3
CW = 8
G = 4
WMAIN = 384
NCHUNKS = WMAIN // CW
WTAIL = NWIN - NW * WMAIN
WT_EACH = WTAIL // NW
WT_REM = WTAIL % NW

_mesh = plsc.VectorSubcoreMesh(core_axis_name="c", subcore_axis_name="s")
_sc_params = pltpu.CompilerParams(use_tc_tiling_on_sc=False)


def _zero_acc(acc_sh, zbuf, s):
    def zb(i, carry):
        zbuf[i] = jnp.zeros((F,), jnp.float32)
        return carry

    lax.fori_loop(0, ZB, zb, 0)
    for j in range(RPT // ZB):
        pltpu.sync_copy(zbuf, acc_sh.at[pl.ds(s * RPT + j * ZB, ZB)])


def _writeback(acc_sh, p0_hbm, p1_hbm, c, s):
    @pl.when(c == 0)
    def _():
        pltpu.sync_copy(acc_sh.at[pl.ds(s * RPT, RPT)],
                        p0_hbm.at[pl.ds(s * RPT, RPT)])

    @pl.when(c == 1)
    def _():
        pltpu.sync_copy(acc_sh.at[pl.ds(s * RPT, RPT)],
                        p1_hbm.at[pl.ds(s * RPT, RPT)])


def _sc_layer_body(g_hbm, src2_hbm, dst2_hbm, p0_hbm, p1_hbm,
                   acc_sh, zbuf, sidx, didx, rows, isem, gsem, ssem):
    c = lax.axis_index("c")
    s = lax.axis_index("s")
    wid = s * NC + c

    _zero_acc(acc_sh, zbuf, s)
    plsc.subcore_barrier()

    base = wid * WMAIN

    def issue_chunk(ch, slot):
        row = base + ch * CW
        pltpu.async_copy(src2_hbm.at[pl.ds(row, CW)], sidx.at[slot],
                         isem.at[slot])
        pltpu.async_copy(dst2_hbm.at[pl.ds(row, CW)], didx.at[slot],
                         isem.at[slot])

    def wait_chunk(slot):
        pltpu.make_async_copy(src2_hbm.at[pl.ds(0, CW)], sidx.at[slot],
                              isem.at[slot]).wait()
        pltpu.make_async_copy(dst2_hbm.at[pl.ds(0, CW)], didx.at[slot],
                              isem.at[slot]).wait()

    def issue_gather(slot, r, b):
        pltpu.async_copy(g_hbm.at[sidx.at[slot, r]], rows.at[b], gsem.at[b])

    def wait_gather(slot, r, b):
        pltpu.make_async_copy(g_hbm.at[sidx.at[slot, r]], rows.at[b],
                              gsem.at[b]).wait()

    def issue_scatter(slot, r, b):
        pltpu.async_copy(rows.at[b], acc_sh.at[didx.at[slot, r]],
                         ssem.at[b], add=True)

    def wait_scatter(b):
        pltpu.make_async_copy(rows.at[b], acc_sh.at[pl.ds(0, WIN)],
                              ssem.at[b]).wait()

    issue_chunk(0, 0)
    issue_chunk(1, 1)
    wait_chunk(0)
    for j in range(G):
        issue_gather(0, j, j)

    def body(i, carry):
        for p in range(NCH * CW):
            k, r = divmod(p, CW)
            w = i * (NCH * CW) + p
            if (p + G) % CW == 0:
                @pl.when(w + G < WMAIN)
                def _():
                    wait_chunk(((p + G) // CW) % NCH)
            wait_gather(k % NCH, r, p % NB)
            issue_scatter(k % NCH, r, p % NB)
            v = w + G

            @pl.when(v < WMAIN)
            def _():
                vb = (p + G) % NB

                @pl.when(v >= NB)
                def _():
                    wait_scatter(vb)

                issue_gather(((p + G) // CW) % NCH, (p + G) % CW, vb)

            if p % CW == CW - 1:
                ch = i * NCH + k + 2

                @pl.when(ch < NCHUNKS)
                def _():
                    issue_chunk(ch, (k + 2) % NCH)
        return carry

    lax.fori_loop(0, NCHUNKS // NCH, body, 0)

    for b in range(NB):
        wait_scatter(b)

    tbase = NW * WMAIN + wid * WT_EACH
    for t in range(WT_EACH):
        pltpu.sync_copy(src2_hbm.at[tbase + t], sidx.at[0, 0])
        pltpu.sync_copy(dst2_hbm.at[tbase + t], didx.at[0, 0])
        pltpu.sync_copy(g_hbm.at[sidx.at[0, 0]], rows.at[0])
        pltpu.sync_copy(rows.at[0], acc_sh.at[didx.at[0, 0]], add=True)

    @pl.when(wid < WT_REM)
    def _():
        row = NW * WMAIN + NW * WT_EACH + wid
        pltpu.sync_copy(src2_hbm.at[row], sidx.at[0, 0])
        pltpu.sync_copy(dst2_hbm.at[row], didx.at[0, 0])
        pltpu.sync_copy(g_hbm.at[sidx.at[0, 0]], rows.at[0])
        pltpu.sync_copy(rows.at[0], acc_sh.at[didx.at[0, 0]], add=True)

    plsc.subcore_barrier()
    _writeback(acc_sh, p0_hbm, p1_hbm, c, s)


def _sc_degree_body(dst2_hbm, p0_hbm, p1_hbm,
                    acc_sh, zbuf, didx, ones_b, isem, ssem):
    c = lax.axis_index("c")
    s = lax.axis_index("s")
    wid = s * NC + c

    _zero_acc(acc_sh, zbuf, s)

    def ob(i, carry):
        ones_b[i] = jnp.ones((F,), jnp.float32)
        return carry

    lax.fori_loop(0, WIN, ob, 0)
    plsc.subcore_barrier()

    base = wid * WMAIN

    def issue_chunk(ch, slot):
        row = base + ch * CW
        pltpu.async_copy(dst2_hbm.at[pl.ds(row, CW)], didx.at[slot],
                         isem.at[slot])

    def wait_chunk(slot):
        pltpu.make_async_copy(dst2_hbm.at[pl.ds(0, CW)], didx.at[slot],
                              isem.at[slot]).wait()

    def wait_scatter(b):
        pltpu.make_async_copy(ones_b, acc_sh.at[pl.ds(0, WIN)],
                              ssem.at[b]).wait()

    issue_chunk(0, 0)
    issue_chunk(1, 1)

    def body(i, carry):
        for p in range(NCH * CW):
            k, r = divmod(p, CW)
            w = i * (NCH * CW) + p
            if p % CW == 0:
                wait_chunk(k % NCH)

            @pl.when(w >= NB)
            def _():
                wait_scatter(p % NB)

            pltpu.async_copy(ones_b, acc_sh.at[didx.at[k % NCH, r]],
                             ssem.at[p % NB], add=True)
            if p % CW == CW - 1:
                ch = i * NCH + k + 2

                @pl.when(ch < NCHUNKS)
                def _():
                    issue_chunk(ch, (k + 2) % NCH)
        return carry

    lax.fori_loop(0, NCHUNKS // NCH, body, 0)
    for b in range(NB):
        wait_scatter(b)

    tbase = NW * WMAIN + wid * WT_EACH
    for t in range(WT_EACH):
        pltpu.sync_copy(dst2_hbm.at[tbase + t], didx.at[0, 0])
        pltpu.sync_copy(ones_b, acc_sh.at[didx.at[0, 0]], add=True)

    @pl.when(wid < WT_REM)
    def _():
        row = NW * WMAIN + NW * WT_EACH + wid
        pltpu.sync_copy(dst2_hbm.at[row], didx.at[0, 0])
        pltpu.sync_copy(ones_b, acc_sh.at[didx.at[0, 0]], add=True)

    plsc.subcore_barrier()
    _writeback(acc_sh, p0_hbm, p1_hbm, c, s)


def _sc_degree(dst2):
    return pl.kernel(
        _sc_degree_body,
        out_type=(jax.ShapeDtypeStruct((N_PAD, F), jnp.float32),
                  jax.ShapeDtypeStruct((N_PAD, F), jnp.float32)),
        mesh=_mesh,
        compiler_params=_sc_params,
        scratch_types=[
            pltpu.VMEM_SHARED((N_PAD, F), jnp.float32),
            pltpu.VMEM((ZB, F), jnp.float32),
            pltpu.VMEM((NCH, CW, WIN), jnp.int32),
            pltpu.VMEM((WIN, F), jnp.float32),
            pltpu.SemaphoreType.DMA((NCH,)),
            pltpu.SemaphoreType.DMA((NB,)),
        ],
    )(dst2)


def _sc_layer(g, src2, dst2):
    return pl.kernel(
        _sc_layer_body,
        out_type=(jax.ShapeDtypeStruct((N_PAD, F), jnp.float32),
                  jax.ShapeDtypeStruct((N_PAD, F), jnp.float32)),
        mesh=_mesh,
        compiler_params=_sc_params,
        scratch_types=[
            pltpu.VMEM_SHARED((N_PAD, F), jnp.float32),
            pltpu.VMEM((ZB, F), jnp.float32),
            pltpu.VMEM((NCH, CW, WIN), jnp.int32),
            pltpu.VMEM((NCH, CW, WIN), jnp.int32),
            pltpu.VMEM((NB, WIN, F), jnp.float32),
            pltpu.SemaphoreType.DMA((NCH,)),
            pltpu.SemaphoreType.DMA((NB,)),
            pltpu.SemaphoreType.DMA((NB,)),
        ],
    )(g, src2, dst2)


_BLK = N_PAD // 16


def _full(s):
    return pl.BlockSpec(s, lambda i: (0,) * len(s))


def _rows(w=F):
    return pl.BlockSpec((_BLK, w), lambda i: (i, 0))


def _mlp_body(x_ref, w1_ref, b1_ref, w2_ref, b2_ref, w3_ref, b3_ref, o_ref):
    h = jnp.tanh(jnp.dot(x_ref[...], w1_ref[...],
                         preferred_element_type=jnp.float32) + b1_ref[...])
    h = jnp.tanh(jnp.dot(h, w2_ref[...],
                         preferred_element_type=jnp.float32) + b2_ref[...])
    o_ref[...] = jnp.dot(h, w3_ref[...],
                         preferred_element_type=jnp.float32) + b3_ref[...]


def _mlp(x, W1, b1, W2, b2, W3, b3):
    BLK = 2000
    return pl.pallas_call(
        _mlp_body,
        grid=(N_NODES // BLK,),
        in_specs=[
            pl.BlockSpec((BLK, IN_DIM), lambda i: (i, 0)),
            _full((IN_DIM, 64)), _full((64,)),
            _full((64, 32)), _full((32,)),
            _full((32, F)), _full((F,)),
        ],
        out_specs=pl.BlockSpec((BLK, F), lambda i: (i, 0)),
        out_shape=jax.ShapeDtypeStruct((N_NODES, F), jnp.float32),
    )(x, W1, b1, W2, b2, W3, b3)


def _tc_pre_body(d0, d1, h0, w, dinv_o, g_o):
    dinv = lax.rsqrt(d0[...] + d1[...] + 1.0)
    dinv_o[...] = dinv
    g_o[...] = dinv * jnp.dot(h0[...], w[...],
                              preferred_element_type=jnp.float32)


def _tc_pre(d0, d1, h0p, W0):
    return pl.pallas_call(
        _tc_pre_body,
        grid=(N_PAD // _BLK,),
        in_specs=[_rows(), _rows(), _rows(), _full((F, F))],
        out_specs=[_rows(), _rows()],
        out_shape=[jax.ShapeDtypeStruct((N_PAD, F), jnp.float32),
                   jax.ShapeDtypeStruct((N_PAD, F), jnp.float32)],
    )(d0, d1, h0p, W0)


def _tc_mid_body(p0, p1, g, dinv, b, w, g_o):
    agg = dinv[...] * (p0[...] + p1[...] + g[...])
    h = jnp.tanh(agg + b[...])
    g_o[...] = dinv[...] * jnp.dot(h, w[...],
                                   preferred_element_type=jnp.float32)


def _tc_mid(p0, p1, g, dinv, b, Wn):
    return pl.pallas_call(
        _tc_mid_body,
        grid=(N_PAD // _BLK,),
        in_specs=[_rows(), _rows(), _rows(), _rows(),
                  _full((F,)), _full((F, F))],
        out_specs=_rows(),
        out_shape=jax.ShapeDtypeStruct((N_PAD, F), jnp.float32),
    )(p0, p1, g, dinv, b, Wn)


def _tc_last_body(p0, p1, g, dinv, b, wc, bc, h_o, out_o):
    agg = dinv[...] * (p0[...] + p1[...] + g[...])
    h = jnp.tanh(agg + b[...])
    h_o[...] = h
    out_o[...] = jnp.dot(h, wc[...],
                         preferred_element_type=jnp.float32) + bc[...]


def _tc_last(p0, p1, g, dinv, b, Wc, bc):
    return pl.pallas_call(
        _tc_last_body,
        grid=(N_PAD // _BLK,),
        in_specs=[_rows(), _rows(), _rows(), _rows(),
                  _full((F,)), _full((F, 2)), _full((2,))],
        out_specs=[_rows(), _rows(2)],
        out_shape=[jax.ShapeDtypeStruct((N_PAD, F), jnp.float32),
                   jax.ShapeDtypeStruct((N_PAD, 2), jnp.float32)],
    )(p0, p1, g, dinv, b, Wc, bc)


def kernel(x, edge_index, W1, b1, W2, b2, W3, b3, convW, convb, Wc, bc):
    src2 = edge_index[0].reshape(NWIN, WIN)
    dst2 = edge_index[1].reshape(NWIN, WIN)

    d0, d1 = _sc_degree(dst2)
    h0 = _mlp(x, W1, b1, W2, b2, W3, b3)
    h0p = jnp.pad(h0, ((0, N_PAD - N_NODES), (0, 0)))
    dinv, g = _tc_pre(d0, d1, h0p, convW[0])

    widx = lambda k: 0 if k < 5 else k - 4
    for k in range(10):
        p0, p1 = _sc_layer(g, src2, dst2)
        if k < 9:
            g = _tc_mid(p0, p1, g, dinv, convb[widx(k)], convW[widx(k + 1)])
        else:
            h, out = _tc_last(p0, p1, g, dinv, convb[widx(k)], Wc, bc)

    return (out[:N_NODES], h[:N_NODES])


# speedup vs baseline: 61.2377x; 2.0399x over previous
import jax
import jax.numpy as jnp
from jax import lax
from jax.experimental import pallas as pl
from jax.experimental.pallas import tpu as pltpu
from jax.experimental.pallas import tpu_sc as plsc

N_NODES = 100000
N_PAD = 100096
N_EDGES = 1600000
IN_DIM = 131
F = 16

NC = 2
NS = 16
NW = NC * NS
WIN = 128
NWIN = N_EDGES // WIN
WPW = NWIN // NW
WREM = NWIN % NW
RPT = N_PAD // NS
ZB = RPT // 16

NB = 8
NCH = 3
CW = 8
G = 4
WMAIN = 384
NCHUNKS = WMAIN // CW
WTAIL = NWIN - NW * WMAIN
WT_EACH = WTAIL // NW
WT_REM = WTAIL % NW

_mesh = plsc.VectorSubcoreMesh(core_axis_name="c", subcore_axis_name="s")
_sc_params = pltpu.CompilerParams(use_tc_tiling_on_sc=False)


def _zero_acc(acc_sh, zbuf, zsem, s):
    def zb(i, carry):
        zbuf[i] = jnp.zeros((F,), jnp.float32)
        return carry

    lax.fori_loop(0, ZB, zb, 0)
    for j in range(RPT // ZB):
        pltpu.async_copy(zbuf, acc_sh.at[pl.ds(s * RPT + j * ZB, ZB)], zsem)
    for j in range(RPT // ZB):
        pltpu.make_async_copy(zbuf, acc_sh.at[pl.ds(0, ZB)], zsem).wait()


def _writeback(acc_sh, p0_hbm, p1_hbm, c, s):
    @pl.when(c == 0)
    def _():
        pltpu.sync_copy(acc_sh.at[pl.ds(s * RPT, RPT)],
                        p0_hbm.at[pl.ds(s * RPT, RPT)])

    @pl.when(c == 1)
    def _():
        pltpu.sync_copy(acc_sh.at[pl.ds(s * RPT, RPT)],
                        p1_hbm.at[pl.ds(s * RPT, RPT)])


def _sc_layer_body(g_hbm, src2_hbm, dst2_hbm, p0_hbm, p1_hbm,
                   acc_sh, zbuf, sidx, didx, rows, isem, gsem, ssem, zsem):
    c = lax.axis_index("c")
    s = lax.axis_index("s")
    wid = s * NC + c

    base = wid * WMAIN

    def issue_chunk(ch, slot):
        row = base + ch * CW
        pltpu.async_copy(src2_hbm.at[pl.ds(row, CW)], sidx.at[slot],
                         isem.at[slot])
        pltpu.async_copy(dst2_hbm.at[pl.ds(row, CW)], didx.at[slot],
                         isem.at[slot])

    def wait_chunk(slot):
        pltpu.make_async_copy(src2_hbm.at[pl.ds(0, CW)], sidx.at[slot],
                              isem.at[slot]).wait()
        pltpu.make_async_copy(dst2_hbm.at[pl.ds(0, CW)], didx.at[slot],
                              isem.at[slot]).wait()

    def issue_gather(slot, r, b):
        pltpu.async_copy(g_hbm.at[sidx.at[slot, r]], rows.at[b], gsem.at[b])

    def wait_gather(slot, r, b):
        pltpu.make_async_copy(g_hbm.at[sidx.at[slot, r]], rows.at[b],
                              gsem.at[b]).wait()

    def issue_scatter(slot, r, b):
        pltpu.async_copy(rows.at[b], acc_sh.at[didx.at[slot, r]],
                         ssem.at[b], add=True)

    def wait_scatter(b):
        pltpu.make_async_copy(rows.at[b], acc_sh.at[pl.ds(0, WIN)],
                              ssem.at[b]).wait()

    issue_chunk(0, 0)
    issue_chunk(1, 1)
    _zero_acc(acc_sh, zbuf, zsem, s)
    wait_chunk(0)
    for j in range(G):
        issue_gather(0, j, j)
    plsc.subcore_barrier()

    def body(i, carry):
        for p in range(NCH * CW):
            k, r = divmod(p, CW)
            w = i * (NCH * CW) + p
            if (p + G) % CW == 0:
                @pl.when(w + G < WMAIN)
                def _():
                    wait_chunk(((p + G) // CW) % NCH)
            wait_gather(k % NCH, r, p % NB)
            issue_scatter(k % NCH, r, p % NB)
            v = w + G

            @pl.when(v < WMAIN)
            def _():
                vb = (p + G) % NB

                @pl.when(v >= NB)
                def _():
                    wait_scatter(vb)

                issue_gather(((p + G) // CW) % NCH, (p + G) % CW, vb)

            if p % CW == CW - 1:
                ch = i * NCH + k + 2

                @pl.when(ch < NCHUNKS)
                def _():
                    issue_chunk(ch, (k + 2) % NCH)
        return carry

    lax.fori_loop(0, NCHUNKS // NCH, body, 0)

    for b in range(NB):
        wait_scatter(b)

    tbase = NW * WMAIN + wid * WT_EACH
    for t in range(WT_EACH):
        pltpu.sync_copy(src2_hbm.at[tbase + t], sidx.at[0, 0])
        pltpu.sync_copy(dst2_hbm.at[tbase + t], didx.at[0, 0])
        pltpu.sync_copy(g_hbm.at[sidx.at[0, 0]], rows.at[0])
        pltpu.sync_copy(rows.at[0], acc_sh.at[didx.at[0, 0]], add=True)

    @pl.when(wid < WT_REM)
    def _():
        row = NW * WMAIN + NW * WT_EACH + wid
        pltpu.sync_copy(src2_hbm.at[row], sidx.at[0, 0])
        pltpu.sync_copy(dst2_hbm.at[row], didx.at[0, 0])
        pltpu.sync_copy(g_hbm.at[sidx.at[0, 0]], rows.at[0])
        pltpu.sync_copy(rows.at[0], acc_sh.at[didx.at[0, 0]], add=True)

    plsc.subcore_barrier()
    _writeback(acc_sh, p0_hbm, p1_hbm, c, s)


def _sc_degree_body(dst2_hbm, p0_hbm, p1_hbm,
                    acc_sh, zbuf, didx, ones_b, isem, ssem, zsem):
    c = lax.axis_index("c")
    s = lax.axis_index("s")
    wid = s * NC + c
    base = wid * WMAIN

    def issue_chunk(ch, slot):
        row = base + ch * CW
        pltpu.async_copy(dst2_hbm.at[pl.ds(row, CW)], didx.at[slot],
                         isem.at[slot])

    def wait_chunk(slot):
        pltpu.make_async_copy(dst2_hbm.at[pl.ds(0, CW)], didx.at[slot],
                              isem.at[slot]).wait()

    def wait_scatter(b):
        pltpu.make_async_copy(ones_b, acc_sh.at[pl.ds(0, WIN)],
                              ssem.at[b]).wait()

    issue_chunk(0, 0)
    issue_chunk(1, 1)
    _zero_acc(acc_sh, zbuf, zsem, s)

    def ob(i, carry):
        ones_b[i] = jnp.ones((F,), jnp.float32)
        return carry

    lax.fori_loop(0, WIN, ob, 0)
    plsc.subcore_barrier()

    def body(i, carry):
        for p in range(NCH * CW):
            k, r = divmod(p, CW)
            w = i * (NCH * CW) + p
            if p % CW == 0:
                wait_chunk(k % NCH)

            @pl.when(w >= NB)
            def _():
                wait_scatter(p % NB)

            pltpu.async_copy(ones_b, acc_sh.at[didx.at[k % NCH, r]],
                             ssem.at[p % NB], add=True)
            if p % CW == CW - 1:
                ch = i * NCH + k + 2

                @pl.when(ch < NCHUNKS)
                def _():
                    issue_chunk(ch, (k + 2) % NCH)
        return carry

    lax.fori_loop(0, NCHUNKS // NCH, body, 0)
    for b in range(NB):
        wait_scatter(b)

    tbase = NW * WMAIN + wid * WT_EACH
    for t in range(WT_EACH):
        pltpu.sync_copy(dst2_hbm.at[tbase + t], didx.at[0, 0])
        pltpu.sync_copy(ones_b, acc_sh.at[didx.at[0, 0]], add=True)

    @pl.when(wid < WT_REM)
    def _():
        row = NW * WMAIN + NW * WT_EACH + wid
        pltpu.sync_copy(dst2_hbm.at[row], didx.at[0, 0])
        pltpu.sync_copy(ones_b, acc_sh.at[didx.at[0, 0]], add=True)

    plsc.subcore_barrier()
    _writeback(acc_sh, p0_hbm, p1_hbm, c, s)


def _sc_degree(dst2):
    return pl.kernel(
        _sc_degree_body,
        out_type=(jax.ShapeDtypeStruct((N_PAD, F), jnp.float32),
                  jax.ShapeDtypeStruct((N_PAD, F), jnp.float32)),
        mesh=_mesh,
        compiler_params=_sc_params,
        scratch_types=[
            pltpu.VMEM_SHARED((N_PAD, F), jnp.float32),
            pltpu.VMEM((ZB, F), jnp.float32),
            pltpu.VMEM((NCH, CW, WIN), jnp.int32),
            pltpu.VMEM((WIN, F), jnp.float32),
            pltpu.SemaphoreType.DMA((NCH,)),
            pltpu.SemaphoreType.DMA((NB,)),
            pltpu.SemaphoreType.DMA,
        ],
    )(dst2)


def _sc_layer(g, src2, dst2):
    return pl.kernel(
        _sc_layer_body,
        out_type=(jax.ShapeDtypeStruct((N_PAD, F), jnp.float32),
                  jax.ShapeDtypeStruct((N_PAD, F), jnp.float32)),
        mesh=_mesh,
        compiler_params=_sc_params,
        scratch_types=[
            pltpu.VMEM_SHARED((N_PAD, F), jnp.float32),
            pltpu.VMEM((ZB, F), jnp.float32),
            pltpu.VMEM((NCH, CW, WIN), jnp.int32),
            pltpu.VMEM((NCH, CW, WIN), jnp.int32),
            pltpu.VMEM((NB, WIN, F), jnp.float32),
            pltpu.SemaphoreType.DMA((NCH,)),
            pltpu.SemaphoreType.DMA((NB,)),
            pltpu.SemaphoreType.DMA((NB,)),
            pltpu.SemaphoreType.DMA,
        ],
    )(g, src2, dst2)


ROWS_F = N_PAD // 8
_BLK = ROWS_F // 4


def _full(s):
    return pl.BlockSpec(s, lambda i: (0,) * len(s))


def _rows(w=128):
    return pl.BlockSpec((_BLK, w), lambda i: (i, 0))


def _mlp_body(x_ref, w1_ref, b1_ref, w2_ref, b2_ref, w3_ref, b3_ref, o_ref):
    h = jnp.tanh(jnp.dot(x_ref[...], w1_ref[...],
                         preferred_element_type=jnp.float32) + b1_ref[...])
    h = jnp.tanh(jnp.dot(h, w2_ref[...],
                         preferred_element_type=jnp.float32) + b2_ref[...])
    o_ref[...] = jnp.dot(h, w3_ref[...],
                         preferred_element_type=jnp.float32) + b3_ref[...]


def _mlp(x, W1, b1, W2, b2, W3, b3):
    BLK = 2000
    return pl.pallas_call(
        _mlp_body,
        grid=(N_NODES // BLK,),
        in_specs=[
            pl.BlockSpec((BLK, IN_DIM), lambda i: (i, 0)),
            _full((IN_DIM, 64)), _full((64,)),
            _full((64, 32)), _full((32,)),
            _full((32, F)), _full((F,)),
        ],
        out_specs=pl.BlockSpec((BLK, F), lambda i: (i, 0)),
        out_shape=jax.ShapeDtypeStruct((N_NODES, F), jnp.float32),
    )(x, W1, b1, W2, b2, W3, b3)


def _tc_pre_body(d0, d1, h0, w8, dinv_o, g_o):
    dinv = lax.rsqrt(d0[...] + d1[...] + 1.0)
    dinv_o[...] = dinv
    g_o[...] = dinv * jnp.dot(h0[...], w8[...],
                              preferred_element_type=jnp.float32)


def _tc_pre(d0f, d1f, h0f, W8_0):
    return pl.pallas_call(
        _tc_pre_body,
        grid=(ROWS_F // _BLK,),
        in_specs=[_rows(), _rows(), _rows(), _full((128, 128))],
        out_specs=[_rows(), _rows()],
        out_shape=[jax.ShapeDtypeStruct((ROWS_F, 128), jnp.float32),
                   jax.ShapeDtypeStruct((ROWS_F, 128), jnp.float32)],
    )(d0f, d1f, h0f, W8_0)


def _tc_mid_body(p0, p1, g, dinv, b, w8, g_o):
    agg = dinv[...] * (p0[...] + p1[...] + g[...])
    h = jnp.tanh(agg + b[...])
    g_o[...] = dinv[...] * jnp.dot(h, w8[...],
                                   preferred_element_type=jnp.float32)


def _tc_mid(p0f, p1f, gf, dinvf, b128, W8n):
    return pl.pallas_call(
        _tc_mid_body,
        grid=(ROWS_F // _BLK,),
        in_specs=[_rows(), _rows(), _rows(), _rows(),
                  _full((128,)), _full((128, 128))],
        out_specs=_rows(),
        out_shape=jax.ShapeDtypeStruct((ROWS_F, 128), jnp.float32),
    )(p0f, p1f, gf, dinvf, b128, W8n)


def _tc_last_body(p0, p1, g, dinv, b, wc8, bc8, h_o, out_o):
    agg = dinv[...] * (p0[...] + p1[...] + g[...])
    h = jnp.tanh(agg + b[...])
    h_o[...] = h
    out_o[...] = jnp.dot(h, wc8[...],
                         preferred_element_type=jnp.float32) + bc8[...]


def _tc_last(p0f, p1f, gf, dinvf, b128, Wc8, bc8):
    return pl.pallas_call(
        _tc_last_body,
        grid=(ROWS_F // _BLK,),
        in_specs=[_rows(), _rows(), _rows(), _rows(),
                  _full((128,)), _full((128, F)), _full((F,))],
        out_specs=[_rows(), _rows(F)],
        out_shape=[jax.ShapeDtypeStruct((ROWS_F, 128), jnp.float32),
                   jax.ShapeDtypeStruct((ROWS_F, F), jnp.float32)],
    )(p0f, p1f, gf, dinvf, b128, Wc8, bc8)


def kernel(x, edge_index, W1, b1, W2, b2, W3, b3, convW, convb, Wc, bc):
    src2 = edge_index[0].reshape(NWIN, WIN)
    dst2 = edge_index[1].reshape(NWIN, WIN)

    eye8 = jnp.eye(8, dtype=jnp.float32)
    W8 = jnp.einsum('ab,kij->kaibj', eye8, convW).reshape(6, 128, 128)
    b128 = jnp.tile(convb, (1, 8))
    Wc8 = jnp.einsum('ab,ij->aibj', eye8, Wc).reshape(128, F)
    bc8 = jnp.tile(bc, 8)

    d0, d1 = _sc_degree(dst2)
    h0 = _mlp(x, W1, b1, W2, b2, W3, b3)
    h0f = jnp.pad(h0, ((0, N_PAD - N_NODES), (0, 0))).reshape(ROWS_F, 128)
    dinvf, gf = _tc_pre(d0.reshape(ROWS_F, 128), d1.reshape(ROWS_F, 128),
                        h0f, W8[0])

    widx = lambda k: 0 if k < 5 else k - 4
    for k in range(10):
        p0, p1 = _sc_layer(gf.reshape(N_PAD, F), src2, dst2)
        p0f = p0.reshape(ROWS_F, 128)
        p1f = p1.reshape(ROWS_F, 128)
        if k < 9:
            gf = _tc_mid(p0f, p1f, gf, dinvf, b128[widx(k)], W8[widx(k + 1)])
        else:
            hf, outf = _tc_last(p0f, p1f, gf, dinvf, b128[widx(k)], Wc8, bc8)

    out = outf.reshape(N_PAD, 2)[:N_NODES]
    h = hf.reshape(N_PAD, F)[:N_NODES]
    return (out, h)
